# baseline (speedup 1.0000x reference)
import functools

import jax
import jax.numpy as jnp
from jax.experimental import pallas as pl
from jax.experimental.pallas import tpu as pltpu
from jax.experimental.pallas import tpu_sc as plsc

_FEATURE_DIM = 128
_NROWS = 9
_NLANES = 16


def _feature_kernel(num_hbm, table_hbm, out_hbm, num_v, table_v, sem_n, sem_t):
    cp_n = pltpu.make_async_copy(num_hbm, num_v.at[pl.ds(0, 1)], sem_n)
    cp_t = pltpu.make_async_copy(table_hbm, table_v, sem_t)
    cp_n.start()
    cp_t.start()
    cp_n.wait()
    n = num_v[...][0]
    idx = jnp.int32(0)
    for b in (1, 2, 3, 4, 8, 16, 32, 64):
        idx = idx + jnp.where(n >= b, jnp.int32(1), jnp.int32(0))
    cp_t.wait()
    pltpu.sync_copy(table_v.at[idx], out_hbm)


def kernel(num, table):
    num_vec = jnp.asarray(num, dtype=jnp.int32).reshape((1,))
    mesh = plsc.VectorSubcoreMesh(
        core_axis_name="c", subcore_axis_name="s", num_cores=1, num_subcores=1)
    run = functools.partial(
        pl.kernel,
        out_type=jax.ShapeDtypeStruct((_FEATURE_DIM,), jnp.float32),
        mesh=mesh,
        scratch_types=[
            pltpu.VMEM((_NLANES,), jnp.int32),
            pltpu.VMEM((_NROWS, _FEATURE_DIM), jnp.float32),
            pltpu.SemaphoreType.DMA,
            pltpu.SemaphoreType.DMA,
        ],
    )(_feature_kernel)
    return run(num_vec, table)


# speedup vs baseline: 1.0968x; 1.0968x over previous
import functools

import jax
import jax.numpy as jnp
from jax.experimental import pallas as pl
from jax.experimental.pallas import tpu as pltpu
from jax.experimental.pallas import tpu_sc as plsc

_FEATURE_DIM = 128
_NROWS = 9


def _feature_kernel(num_hbm, table_hbm, out_hbm, num_s, table_vs, sem_n, sem_t):
    cp_n = pltpu.make_async_copy(num_hbm, num_s, sem_n)
    cp_t = pltpu.make_async_copy(table_hbm, table_vs, sem_t)
    cp_n.start()
    cp_t.start()
    cp_n.wait()
    n = num_s[0]
    idx = jnp.int32(0)
    for b in (1, 2, 3, 4, 8, 16, 32, 64):
        idx = idx + jnp.where(n >= b, jnp.int32(1), jnp.int32(0))
    cp_t.wait()
    pltpu.sync_copy(table_vs.at[idx], out_hbm)


def kernel(num, table):
    num_vec = jnp.asarray(num, dtype=jnp.int32).reshape((1,))
    mesh = plsc.ScalarSubcoreMesh(axis_name="c", num_cores=1)
    run = functools.partial(
        pl.kernel,
        out_type=jax.ShapeDtypeStruct((_FEATURE_DIM,), jnp.float32),
        mesh=mesh,
        compiler_params=pltpu.CompilerParams(skip_device_barrier=True),
        scratch_types=[
            pltpu.SMEM((1,), jnp.int32),
            pltpu.VMEM_SHARED((_NROWS, _FEATURE_DIM), jnp.float32),
            pltpu.SemaphoreType.DMA,
            pltpu.SemaphoreType.DMA,
        ],
    )(_feature_kernel)
    return run(num_vec, table)
